# baseline (speedup 1.0000x reference)
import functools

import jax
import jax.numpy as jnp
from jax import lax
from jax.experimental import pallas as pl
from jax.experimental.pallas import tpu as pltpu
from jax.experimental.pallas import tpu_sc as plsc

S = 256
D = 768
L = 48
K = 4
NBUF = 3

_NC = 2
_NS = 16
_NW = _NC * _NS
_TPW = S // _NW
_NV = L // 16


def _logits_kernel(x_ref, gate_w_ref, o_ref):
    o_ref[...] = jax.lax.dot_general(
        x_ref[...], gate_w_ref[...], (((1,), (1,)), ((), ())),
        preferred_element_type=jnp.float32)


def _gather16(v, idx):
    dn = lax.GatherDimensionNumbers(
        offset_dims=(), collapsed_slice_dims=(0,), start_index_map=(0,))
    return lax.gather(v, idx[:, None], dn, (1,),
                      mode=lax.GatherScatterMode.PROMISE_IN_BOUNDS)


def _bcast_lane(v, j):
    return _gather16(v, jnp.full((16,), j, jnp.int32))


def _allmax(v):
    iota = lax.iota(jnp.int32, 16)
    for sh in (8, 4, 2, 1):
        v = jnp.maximum(v, _gather16(v, iota ^ sh))
    return v


def _allmin_i32(v):
    iota = lax.iota(jnp.int32, 16)
    for sh in (8, 4, 2, 1):
        v = jnp.minimum(v, _gather16(v, iota ^ sh))
    return v


def _allsum(v):
    iota = lax.iota(jnp.int32, 16)
    for sh in (8, 4, 2, 1):
        v = v + _gather16(v, iota ^ sh)
    return v


def _router_body(logits_hbm, g_hbm, lg_v, g_v):
    wid = lax.axis_index("s") * _NC + lax.axis_index("c")
    base = wid * _TPW
    pltpu.sync_copy(logits_hbm.at[pl.ds(base, _TPW)], lg_v)
    iota = lax.iota(jnp.int32, 16)
    neg_inf = jnp.float32(-jnp.inf)
    big = jnp.int32(1 << 20)
    for t in range(_TPW):
        work = [lg_v[t, pl.ds(16 * c, 16)] for c in range(_NV)]
        idxs = []
        vals = []
        for _ in range(K):
            m = work[0]
            for c in range(1, _NV):
                m = jnp.maximum(m, work[c])
            bmax = _allmax(m)
            cand = jnp.full((16,), big, jnp.int32)
            for c in range(_NV):
                cand = jnp.minimum(
                    cand, jnp.where(work[c] >= bmax, iota + 16 * c, big))
            bidx = _allmin_i32(cand)
            idxs.append(bidx)
            vals.append(bmax)
            for c in range(_NV):
                work[c] = jnp.where(iota + 16 * c == bidx, neg_inf, work[c])
        tv = jnp.full((16,), neg_inf, jnp.float32)
        for j in range(K):
            tv = jnp.where(iota == j, vals[j], tv)
        e = jnp.exp(tv - vals[0])
        tot = _allsum(e)
        p = e / tot
        ps = [_bcast_lane(p, j) for j in range(K)]
        for c in range(_NV):
            row = jnp.zeros((16,), jnp.float32)
            for j in range(K):
                row = jnp.where(iota + 16 * c == idxs[j], ps[j], row)
            g_v[t, pl.ds(16 * c, 16)] = row
    pltpu.sync_copy(g_v, g_hbm.at[pl.ds(base, _TPW)])


def _make_router():
    mesh = plsc.VectorSubcoreMesh(core_axis_name="c", subcore_axis_name="s")
    return pl.kernel(
        _router_body,
        mesh=mesh,
        out_type=jax.ShapeDtypeStruct((S, L), jnp.float32),
        scratch_types=[
            pltpu.VMEM((_TPW, L), jnp.float32),
            pltpu.VMEM((_TPW, L), jnp.float32),
        ],
    )


def _combine_kernel(x_ref, g_ref, w_hbm, out_w_ref, out_b_ref,
                    ln_g_ref, ln_b_ref, o_ref, acc_ref, wbuf, sem):
    l = pl.program_id(0)

    def _copy(i, slot):
        return pltpu.make_async_copy(w_hbm.at[i], wbuf.at[slot], sem.at[slot])

    @pl.when(l == 0)
    def _prologue():
        for i in range(NBUF):
            _copy(i, i).start()
        acc_ref[...] = jnp.zeros((S, D), jnp.float32)

    slot = jax.lax.rem(l, NBUF)
    _copy(l, slot).wait()

    lane = jax.lax.broadcasted_iota(jnp.int32, (S, L), 1)
    g_col = jnp.sum(jnp.where(lane == l, g_ref[...], 0.0), axis=-1,
                    keepdims=True)
    y = jax.lax.dot_general(
        x_ref[...], wbuf[slot], (((1,), (0,)), ((), ())),
        preferred_element_type=jnp.float32,
        precision=jax.lax.Precision.DEFAULT)
    acc_ref[...] += g_col * y

    @pl.when(l + NBUF < L)
    def _refill():
        _copy(l + NBUF, slot).start()

    @pl.when(l == L - 1)
    def _epilogue():
        x = x_ref[...]
        h = x + jax.lax.dot_general(
            acc_ref[...], out_w_ref[...], (((1,), (1,)), ((), ())),
            preferred_element_type=jnp.float32) + out_b_ref[...]
        mean = jnp.mean(h, axis=-1, keepdims=True)
        c = h - mean
        var = jnp.mean(c * c, axis=-1, keepdims=True)
        o_ref[...] = c * jax.lax.rsqrt(var + 1e-5) * ln_g_ref[...] + ln_b_ref[...]


@functools.partial(jax.jit, static_argnames=())
def kernel(x, gate_w, lattice_weights, out_w, out_b, ln_gamma, ln_beta):
    x2 = x.reshape(S, D)
    logits = pl.pallas_call(
        _logits_kernel,
        in_specs=[
            pl.BlockSpec((S, D), lambda: (0, 0)),
            pl.BlockSpec((L, D), lambda: (0, 0)),
        ],
        out_specs=pl.BlockSpec((S, L), lambda: (0, 0)),
        out_shape=jax.ShapeDtypeStruct((S, L), jnp.float32),
    )(x2, gate_w)
    g = _make_router()(logits)
    out = pl.pallas_call(
        _combine_kernel,
        grid=(L,),
        in_specs=[
            pl.BlockSpec((S, D), lambda l: (0, 0)),
            pl.BlockSpec((S, L), lambda l: (0, 0)),
            pl.BlockSpec(memory_space=pl.ANY),
            pl.BlockSpec((D, D), lambda l: (0, 0)),
            pl.BlockSpec((1, D), lambda l: (0, 0)),
            pl.BlockSpec((1, D), lambda l: (0, 0)),
            pl.BlockSpec((1, D), lambda l: (0, 0)),
        ],
        out_specs=pl.BlockSpec((S, D), lambda l: (0, 0)),
        out_shape=jax.ShapeDtypeStruct((S, D), jnp.float32),
        scratch_shapes=[
            pltpu.VMEM((S, D), jnp.float32),
            pltpu.VMEM((NBUF, D, D), jnp.float32),
            pltpu.SemaphoreType.DMA((NBUF,)),
        ],
        compiler_params=pltpu.CompilerParams(
            dimension_semantics=("arbitrary",),
        ),
    )(x2, g, lattice_weights, out_w, out_b.reshape(1, D),
      ln_gamma.reshape(1, D), ln_beta.reshape(1, D))
    return out.reshape(1, S, D)


# speedup vs baseline: 1.4016x; 1.4016x over previous
import functools

import jax
import jax.numpy as jnp
from jax.experimental import pallas as pl
from jax.experimental.pallas import tpu as pltpu

S = 256
D = 768
L = 48
K = 4
NBUF = 4


def _fused_kernel(x_ref, gate_w_ref, w_hbm, out_w_ref, out_b_ref,
                  ln_g_ref, ln_b_ref, o_ref, g_ref, acc_ref, wbuf, sem):
    l = pl.program_id(0)

    def _copy_a(i, slot):
        return pltpu.make_async_copy(
            w_hbm.at[i, 0:D // 2], wbuf.at[slot, 0:D // 2], sem.at[slot, 0])

    def _copy_b(i, slot):
        return pltpu.make_async_copy(
            w_hbm.at[i, D // 2:D], wbuf.at[slot, D // 2:D], sem.at[slot, 1])

    def _start(i, slot):
        _copy_a(i, slot).start()
        _copy_b(i, slot).start()

    def _wait(i, slot):
        _copy_a(i, slot).wait()
        _copy_b(i, slot).wait()

    @pl.when(l == 0)
    def _prologue():
        for i in range(NBUF):
            _start(i, i)

        x = x_ref[...]
        logits = jax.lax.dot_general(
            x, gate_w_ref[...], (((1,), (1,)), ((), ())),
            preferred_element_type=jnp.float32)
        lane = jax.lax.broadcasted_iota(jnp.int32, (S, L), 1)
        work = logits
        neg_inf = jnp.float32(-jnp.inf)
        vals = []
        sels = []
        for _ in range(K):
            m = jnp.max(work, axis=-1, keepdims=True)
            is_m = work >= m
            first = jnp.min(jnp.where(is_m, lane, L), axis=-1,
                            keepdims=True)
            sel = lane == first
            vals.append(m)
            sels.append(sel)
            work = jnp.where(sel, neg_inf, work)
        v = jnp.concatenate(vals, axis=-1)
        mx = jnp.max(v, axis=-1, keepdims=True)
        e = jnp.exp(v - mx)
        p = e / jnp.sum(e, axis=-1, keepdims=True)
        g = jnp.zeros((S, L), jnp.float32)
        for j in range(K):
            g = g + jnp.where(sels[j], p[:, j:j + 1], 0.0)
        g_ref[...] = g
        acc_ref[...] = jnp.zeros((S, D), jnp.float32)

    slot = jax.lax.rem(l, NBUF)
    _wait(l, slot)

    lane = jax.lax.broadcasted_iota(jnp.int32, (S, L), 1)
    g_col = jnp.sum(jnp.where(lane == l, g_ref[...], 0.0), axis=-1,
                    keepdims=True)
    y = jax.lax.dot_general(
        x_ref[...], wbuf[slot], (((1,), (0,)), ((), ())),
        preferred_element_type=jnp.float32,
        precision=jax.lax.Precision.DEFAULT)
    acc_ref[...] += g_col * y

    @pl.when(l + NBUF < L)
    def _refill():
        _start(l + NBUF, slot)

    @pl.when(l == L - 1)
    def _epilogue():
        x = x_ref[...]
        h = x + jax.lax.dot_general(
            acc_ref[...], out_w_ref[...], (((1,), (1,)), ((), ())),
            preferred_element_type=jnp.float32) + out_b_ref[...]
        mean = jnp.mean(h, axis=-1, keepdims=True)
        c = h - mean
        var = jnp.mean(c * c, axis=-1, keepdims=True)
        o_ref[...] = c * jax.lax.rsqrt(var + 1e-5) * ln_g_ref[...] + ln_b_ref[...]


@functools.partial(jax.jit, static_argnames=())
def kernel(x, gate_w, lattice_weights, out_w, out_b, ln_gamma, ln_beta):
    x2 = x.reshape(S, D)
    out = pl.pallas_call(
        _fused_kernel,
        grid=(L,),
        in_specs=[
            pl.BlockSpec((S, D), lambda l: (0, 0)),
            pl.BlockSpec((L, D), lambda l: (0, 0)),
            pl.BlockSpec(memory_space=pl.ANY),
            pl.BlockSpec((D, D), lambda l: (0, 0)),
            pl.BlockSpec((1, D), lambda l: (0, 0)),
            pl.BlockSpec((1, D), lambda l: (0, 0)),
            pl.BlockSpec((1, D), lambda l: (0, 0)),
        ],
        out_specs=pl.BlockSpec((S, D), lambda l: (0, 0)),
        out_shape=jax.ShapeDtypeStruct((S, D), jnp.float32),
        scratch_shapes=[
            pltpu.VMEM((S, L), jnp.float32),
            pltpu.VMEM((S, D), jnp.float32),
            pltpu.VMEM((NBUF, D, D), jnp.float32),
            pltpu.SemaphoreType.DMA((NBUF, 2)),
        ],
        compiler_params=pltpu.CompilerParams(
            dimension_semantics=("arbitrary",),
        ),
    )(x2, gate_w, lattice_weights, out_w, out_b.reshape(1, D),
      ln_gamma.reshape(1, D), ln_beta.reshape(1, D))
    return out.reshape(1, S, D)
